# baseline (speedup 1.0000x reference)
import jax
import jax.numpy as jnp
from jax.experimental import pallas as pl
from jax.experimental.pallas import tpu as pltpu

BATCH = 16384
MAX_LEN = 150
ROWS_PER_BLOCK = 256


def _body(x_ref, ids_ref, mask_ref, type_ref):
    x = x_ref[...]
    ids_ref[...] = x
    mask_ref[...] = jnp.where(x == 0, 0, 1).astype(jnp.int32)
    type_ref[...] = jnp.zeros_like(x)


def kernel(inputs):
    grid = (BATCH // ROWS_PER_BLOCK,)
    spec = pl.BlockSpec((ROWS_PER_BLOCK, MAX_LEN), lambda i: (i, 0))
    out_shape = jax.ShapeDtypeStruct((BATCH, MAX_LEN), jnp.int32)
    ids, mask, type_ids = pl.pallas_call(
        _body,
        grid=grid,
        in_specs=[spec],
        out_specs=[spec, spec, spec],
        out_shape=[out_shape, out_shape, out_shape],
        compiler_params=pltpu.CompilerParams(
            dimension_semantics=("parallel",),
        ),
    )(inputs)
    return (ids, mask, type_ids)


# speedup vs baseline: 1.3599x; 1.3599x over previous
import jax
import jax.numpy as jnp
from jax.experimental import pallas as pl
from jax.experimental.pallas import tpu as pltpu

BATCH = 16384
MAX_LEN = 150
R = 2048
NCH = BATCH // R
NBUF = 4


def _body(in_hbm, ids_hbm, mask_hbm, type_hbm,
          ibuf, mbuf, zbuf, in_sem, ids_sem, mask_sem, z_sem):
    def in_dma(i, s):
        return pltpu.make_async_copy(
            in_hbm.at[pl.ds(i * R, R)], ibuf.at[s], in_sem.at[s])

    def ids_dma(i, s):
        return pltpu.make_async_copy(
            ibuf.at[s], ids_hbm.at[pl.ds(i * R, R)], ids_sem.at[s])

    def mask_dma(i, s):
        return pltpu.make_async_copy(
            mbuf.at[s], mask_hbm.at[pl.ds(i * R, R)], mask_sem.at[s])

    def z_dma(i):
        return pltpu.make_async_copy(
            zbuf, type_hbm.at[pl.ds(i * R, R)], z_sem.at[i])

    zbuf[...] = jnp.zeros_like(zbuf)
    for k in range(NBUF - 1):
        in_dma(k, k).start()
    for i in range(NCH):
        s = i % NBUF
        z_dma(i).start()
        j = i + NBUF - 1
        if j < NCH:
            sp = j % NBUF
            if i >= 1:
                ids_dma(i - 1, sp).wait()
            in_dma(j, sp).start()
        in_dma(i, s).wait()
        ids_dma(i, s).start()
        if i >= NBUF:
            mask_dma(i - NBUF, s).wait()
        mbuf[s] = jnp.where(ibuf[s] == 0, 0, 1).astype(jnp.int32)
        mask_dma(i, s).start()
    for i in range(NCH - NBUF, NCH):
        ids_dma(i, i % NBUF).wait()
        mask_dma(i, i % NBUF).wait()
    for i in range(NCH):
        z_dma(i).wait()


def kernel(inputs):
    out_shape = jax.ShapeDtypeStruct((BATCH, MAX_LEN), jnp.int32)
    any_spec = pl.BlockSpec(memory_space=pl.ANY)
    ids, mask, type_ids = pl.pallas_call(
        _body,
        in_specs=[any_spec],
        out_specs=[any_spec, any_spec, any_spec],
        out_shape=[out_shape, out_shape, out_shape],
        scratch_shapes=[
            pltpu.VMEM((NBUF, R, MAX_LEN), jnp.int32),
            pltpu.VMEM((NBUF, R, MAX_LEN), jnp.int32),
            pltpu.VMEM((R, MAX_LEN), jnp.int32),
            pltpu.SemaphoreType.DMA((NBUF,)),
            pltpu.SemaphoreType.DMA((NBUF,)),
            pltpu.SemaphoreType.DMA((NBUF,)),
            pltpu.SemaphoreType.DMA((NCH,)),
        ],
    )(inputs)
    return (ids, mask, type_ids)
